# baseline (speedup 1.0000x reference)
import functools

import jax
import jax.numpy as jnp
from jax import lax
from jax.experimental import pallas as pl
from jax.experimental.pallas import tpu as pltpu
from jax.experimental.pallas import tpu_sc as plsc

N = 10000
E = 320000
D = 128
NC = 2
NS = 16
NW = NC * NS
K = 128
STEPS = 80
PSTEPS = 40
EPW = K * STEPS
PAD_E = NW * EPW
ACC_ROWS = 10240
ZROWS = ACC_ROWS // NS

def _sc_mesh():
    return plsc.VectorSubcoreMesh(core_axis_name="c", subcore_axis_name="s")


def _deg_body(dst_hbm, ones_hbm, zeros_hbm, out_hbm, dst_all, ones_v, acc_sh, sem):
    cid = lax.axis_index("c")
    sid = lax.axis_index("s")
    wid = cid * NS + sid
    pltpu.sync_copy(zeros_hbm, acc_sh.at[pl.ds(sid * ZROWS, ZROWS)])
    pltpu.sync_copy(ones_hbm, ones_v)
    pltpu.sync_copy(dst_hbm.at[wid], dst_all)
    plsc.subcore_barrier()

    @pl.loop(0, STEPS)
    def _(i):
        pltpu.sync_copy(ones_v, acc_sh.at[dst_all.at[i]], add=True)

    plsc.subcore_barrier()
    pltpu.sync_copy(acc_sh.at[pl.ds(sid * ZROWS, ZROWS)],
                    out_hbm.at[cid, pl.ds(sid * ZROWS, ZROWS)])


def _deg_kernel(dst3, ones, zerosD):
    return pl.kernel(
        _deg_body, mesh=_sc_mesh(),
        out_type=jax.ShapeDtypeStruct((NC, ACC_ROWS, D), jnp.float32),
        scratch_types=[
            pltpu.VMEM((STEPS, K), jnp.int32),
            pltpu.VMEM((K, D), jnp.float32),
            pltpu.VMEM_SHARED((ACC_ROWS, D), jnp.float32),
            pltpu.SemaphoreType.DMA,
        ],
    )(dst3, ones, zerosD)


def _segsum_body(g_hbm, src_hbm, dst_hbm, zeros_hbm, out_hbm,
                 src_p, dst_p, rows_a, rows_b, acc_sh, sem_a, sem_b):
    cid = lax.axis_index("c")
    sid = lax.axis_index("s")
    wid = cid * NS + sid
    pltpu.sync_copy(zeros_hbm, acc_sh.at[pl.ds(sid * ZROWS, ZROWS)])
    plsc.subcore_barrier()

    @pl.loop(0, 2)
    def _(p):
        pltpu.sync_copy(src_hbm.at[wid, pl.ds(p * PSTEPS, PSTEPS)], src_p)
        pltpu.sync_copy(dst_hbm.at[wid, pl.ds(p * PSTEPS, PSTEPS)], dst_p)
        @pl.loop(0, PSTEPS)
        def _(i):
            pltpu.async_copy(g_hbm.at[src_p.at[i]], rows_a, sem_a).wait()
            pltpu.sync_copy(rows_a, acc_sh.at[dst_p.at[i]], add=True)

    plsc.subcore_barrier()
    pltpu.sync_copy(acc_sh.at[pl.ds(sid * ZROWS, ZROWS)],
                    out_hbm.at[cid, pl.ds(sid * ZROWS, ZROWS)])


def _segsum_kernel(g, src3, dst3, zerosD):
    return pl.kernel(
        _segsum_body, mesh=_sc_mesh(),
        out_type=jax.ShapeDtypeStruct((NC, ACC_ROWS, D), jnp.float32),
        scratch_types=[
            pltpu.VMEM((PSTEPS, K), jnp.int32),
            pltpu.VMEM((PSTEPS, K), jnp.int32),
            pltpu.VMEM((K, D), jnp.float32),
            pltpu.VMEM((K, D), jnp.float32),
            pltpu.VMEM_SHARED((ACC_ROWS, D), jnp.float32),
            pltpu.SemaphoreType.DMA,
            pltpu.SemaphoreType.DMA,
        ],
    )(g, src3, dst3, zerosD)


_RB = 2000


def _mm_body(x_ref, w_ref, o_ref):
    o_ref[...] = jnp.dot(x_ref[...], w_ref[...],
                         preferred_element_type=jnp.float32)


def _matmul(x, W):
    return pl.pallas_call(
        _mm_body,
        grid=(N // _RB,),
        in_specs=[pl.BlockSpec((_RB, D), lambda i: (i, 0)),
                  pl.BlockSpec((D, D), lambda i: (0, 0))],
        out_specs=pl.BlockSpec((_RB, D), lambda i: (i, 0)),
        out_shape=jax.ShapeDtypeStruct((N, D), jnp.float32),
    )(x, W)


def _scale_body(degp_ref, h_ref, g_ref, dinv_ref):
    deg = degp_ref[0][:, :16] + degp_ref[1][:, :16] + 1.0
    dinv = lax.rsqrt(deg)
    dinv_ref[...] = dinv
    g_ref[...] = h_ref[...] * dinv[:, :1]


def _scale(deg_parts, h):
    return pl.pallas_call(
        _scale_body,
        grid=(N // _RB,),
        in_specs=[pl.BlockSpec((NC, _RB, D), lambda i: (0, i, 0)),
                  pl.BlockSpec((_RB, D), lambda i: (i, 0))],
        out_specs=[pl.BlockSpec((_RB, D), lambda i: (i, 0)),
                   pl.BlockSpec((_RB, 16), lambda i: (i, 0))],
        out_shape=[jax.ShapeDtypeStruct((N, D), jnp.float32),
                   jax.ShapeDtypeStruct((N, 16), jnp.float32)],
    )(deg_parts, h)


def _mid_body(acc_ref, h1_ref, dinv_ref, b1_ref, w2_ref, g2_ref, h2_ref):
    dinv = dinv_ref[...][:, :1]
    out1 = dinv * (acc_ref[0] + acc_ref[1]) \
        + (dinv * dinv) * h1_ref[...] + b1_ref[...]
    h = jnp.maximum(out1, 0.0)
    h2 = jnp.dot(h, w2_ref[...], preferred_element_type=jnp.float32)
    h2_ref[...] = h2
    g2_ref[...] = h2 * dinv


def _mid(acc1, h1, dinv, b1, W2):
    return pl.pallas_call(
        _mid_body,
        grid=(N // _RB,),
        in_specs=[pl.BlockSpec((NC, _RB, D), lambda i: (0, i, 0)),
                  pl.BlockSpec((_RB, D), lambda i: (i, 0)),
                  pl.BlockSpec((_RB, 16), lambda i: (i, 0)),
                  pl.BlockSpec((1, D), lambda i: (0, 0)),
                  pl.BlockSpec((D, D), lambda i: (0, 0))],
        out_specs=[pl.BlockSpec((_RB, D), lambda i: (i, 0)),
                   pl.BlockSpec((_RB, D), lambda i: (i, 0))],
        out_shape=[jax.ShapeDtypeStruct((N, D), jnp.float32),
                   jax.ShapeDtypeStruct((N, D), jnp.float32)],
    )(acc1, h1, dinv, b1, W2)


def _post_body(acc_ref, h2_ref, dinv_ref, b2_ref, out_ref):
    dinv = dinv_ref[...][:, :1]
    out_ref[...] = dinv * (acc_ref[0] + acc_ref[1]) \
        + (dinv * dinv) * h2_ref[...] + b2_ref[...]


def _post(acc2, h2, dinv, b2):
    return pl.pallas_call(
        _post_body,
        grid=(N // _RB,),
        in_specs=[pl.BlockSpec((NC, _RB, D), lambda i: (0, i, 0)),
                  pl.BlockSpec((_RB, D), lambda i: (i, 0)),
                  pl.BlockSpec((_RB, 16), lambda i: (i, 0)),
                  pl.BlockSpec((1, D), lambda i: (0, 0))],
        out_specs=pl.BlockSpec((_RB, D), lambda i: (i, 0)),
        out_shape=jax.ShapeDtypeStruct((N, D), jnp.float32),
    )(acc2, h2, dinv, b2)


def kernel(x, edge_index, W1, b1, W2, b2):
    ei = edge_index.astype(jnp.int32)
    pad = PAD_E - E
    src3 = jnp.concatenate([ei[0], jnp.zeros((pad,), jnp.int32)]) \
        .reshape(NW, STEPS, K)
    pad_dst = N + (jnp.arange(pad, dtype=jnp.int32) % (ACC_ROWS - N))
    dst3 = jnp.concatenate([ei[1], pad_dst]).reshape(NW, STEPS, K)
    ones = jnp.ones((K, D), jnp.float32)
    zerosD = jnp.zeros((ZROWS, D), jnp.float32)
    b1r = b1.reshape(1, D)
    b2r = b2.reshape(1, D)

    deg_parts = _deg_kernel(dst3, ones, zerosD)
    h1 = _matmul(x, W1)
    g1, dinv = _scale(deg_parts, h1)
    acc1 = _segsum_kernel(g1, src3, dst3, zerosD)
    g2, h2 = _mid(acc1, h1, dinv, b1r, W2)
    acc2 = _segsum_kernel(g2, src3, dst3, zerosD)
    return _post(acc2, h2, dinv, b2r)


# speedup vs baseline: 1.6148x; 1.6148x over previous
import functools

import jax
import jax.numpy as jnp
from jax import lax
from jax.experimental import pallas as pl
from jax.experimental.pallas import tpu as pltpu
from jax.experimental.pallas import tpu_sc as plsc

N = 10000
E = 320000
D = 128
NC = 2
NS = 16
NW = NC * NS
K = 80
STEPS = 126
EPW = K * STEPS
PAD_E = NW * EPW
DK = 128
DSTEPS = 80
DPAD_E = NW * DK * DSTEPS
ACC_ROWS = 10240
ZROWS = ACC_ROWS // NS

def _sc_mesh():
    return plsc.VectorSubcoreMesh(core_axis_name="c", subcore_axis_name="s")


def _deg_body(dst_hbm, ones_hbm, zeros_hbm, out_hbm, dst_all, ones_v, acc_sh, sem):
    cid = lax.axis_index("c")
    sid = lax.axis_index("s")
    wid = cid * NS + sid
    pltpu.sync_copy(zeros_hbm, acc_sh.at[pl.ds(sid * ZROWS, ZROWS)])
    pltpu.sync_copy(ones_hbm, ones_v)
    pltpu.sync_copy(dst_hbm.at[wid], dst_all)
    plsc.subcore_barrier()

    @pl.loop(0, DSTEPS)
    def _(i):
        pltpu.sync_copy(ones_v, acc_sh.at[dst_all.at[i]], add=True)

    plsc.subcore_barrier()
    pltpu.sync_copy(acc_sh.at[pl.ds(sid * ZROWS, ZROWS)],
                    out_hbm.at[cid, pl.ds(sid * ZROWS, ZROWS)])


def _deg_kernel(dst3, ones, zerosD):
    return pl.kernel(
        _deg_body, mesh=_sc_mesh(),
        out_type=jax.ShapeDtypeStruct((NC, ACC_ROWS, D), jnp.float32),
        scratch_types=[
            pltpu.VMEM((DSTEPS, DK), jnp.int32),
            pltpu.VMEM((DK, D), jnp.float32),
            pltpu.VMEM_SHARED((ACC_ROWS, D), jnp.float32),
            pltpu.SemaphoreType.DMA,
        ],
    )(dst3, ones, zerosD)


def _segsum_body(g_hbm, src_hbm, dst_hbm, zeros_hbm, out_hbm,
                 src_a, dst_a, src_b, dst_b, rows_a, rows_b, acc_sh,
                 sem_a, sem_b):
    cid = lax.axis_index("c")
    sid = lax.axis_index("s")
    wid = cid * NS + sid
    pltpu.sync_copy(zeros_hbm, acc_sh.at[pl.ds(sid * ZROWS, ZROWS)])
    plsc.subcore_barrier()
    base = wid * EPW
    pltpu.sync_copy(src_hbm.at[pl.ds(base, K)], src_a)
    pltpu.sync_copy(dst_hbm.at[pl.ds(base, K)], dst_a)
    pltpu.async_copy(g_hbm.at[src_a], rows_a, sem_a)

    @pl.loop(0, STEPS, step=2)
    def _(i):
        pltpu.sync_copy(src_hbm.at[pl.ds(base + (i + 1) * K, K)], src_b)
        pltpu.sync_copy(dst_hbm.at[pl.ds(base + (i + 1) * K, K)], dst_b)
        pltpu.async_copy(g_hbm.at[src_b], rows_b, sem_b)
        pltpu.make_async_copy(g_hbm.at[src_a], rows_a, sem_a).wait()
        pltpu.sync_copy(rows_a, acc_sh.at[dst_a], add=True)

        @pl.when(i + 2 < STEPS)
        def _():
            pltpu.sync_copy(src_hbm.at[pl.ds(base + (i + 2) * K, K)], src_a)
            pltpu.sync_copy(dst_hbm.at[pl.ds(base + (i + 2) * K, K)], dst_a)
            pltpu.async_copy(g_hbm.at[src_a], rows_a, sem_a)

        pltpu.make_async_copy(g_hbm.at[src_b], rows_b, sem_b).wait()
        pltpu.sync_copy(rows_b, acc_sh.at[dst_b], add=True)

    plsc.subcore_barrier()
    pltpu.sync_copy(acc_sh.at[pl.ds(sid * ZROWS, ZROWS)],
                    out_hbm.at[cid, pl.ds(sid * ZROWS, ZROWS)])


def _segsum_kernel(g, src, dst, zerosD):
    return pl.kernel(
        _segsum_body, mesh=_sc_mesh(),
        out_type=jax.ShapeDtypeStruct((NC, ACC_ROWS, D), jnp.float32),
        scratch_types=[
            pltpu.VMEM((K,), jnp.int32),
            pltpu.VMEM((K,), jnp.int32),
            pltpu.VMEM((K,), jnp.int32),
            pltpu.VMEM((K,), jnp.int32),
            pltpu.VMEM((K, D), jnp.float32),
            pltpu.VMEM((K, D), jnp.float32),
            pltpu.VMEM_SHARED((ACC_ROWS, D), jnp.float32),
            pltpu.SemaphoreType.DMA,
            pltpu.SemaphoreType.DMA,
        ],
    )(g, src, dst, zerosD)


_RB = 2000


def _mm_body(x_ref, w_ref, o_ref):
    o_ref[...] = jnp.dot(x_ref[...], w_ref[...],
                         preferred_element_type=jnp.float32)


def _matmul(x, W):
    return pl.pallas_call(
        _mm_body,
        grid=(N // _RB,),
        in_specs=[pl.BlockSpec((_RB, D), lambda i: (i, 0)),
                  pl.BlockSpec((D, D), lambda i: (0, 0))],
        out_specs=pl.BlockSpec((_RB, D), lambda i: (i, 0)),
        out_shape=jax.ShapeDtypeStruct((N, D), jnp.float32),
    )(x, W)


def _scale_body(degp_ref, h_ref, g_ref, dinv_ref):
    deg = degp_ref[0][:, :16] + degp_ref[1][:, :16] + 1.0
    dinv = lax.rsqrt(deg)
    dinv_ref[...] = dinv
    g_ref[...] = h_ref[...] * dinv[:, :1]


def _scale(deg_parts, h):
    return pl.pallas_call(
        _scale_body,
        grid=(N // _RB,),
        in_specs=[pl.BlockSpec((NC, _RB, D), lambda i: (0, i, 0)),
                  pl.BlockSpec((_RB, D), lambda i: (i, 0))],
        out_specs=[pl.BlockSpec((_RB, D), lambda i: (i, 0)),
                   pl.BlockSpec((_RB, 16), lambda i: (i, 0))],
        out_shape=[jax.ShapeDtypeStruct((N, D), jnp.float32),
                   jax.ShapeDtypeStruct((N, 16), jnp.float32)],
    )(deg_parts, h)


def _mid_body(acc_ref, h1_ref, dinv_ref, b1_ref, w2_ref, g2_ref, h2_ref):
    dinv = dinv_ref[...][:, :1]
    out1 = dinv * (acc_ref[0] + acc_ref[1]) \
        + (dinv * dinv) * h1_ref[...] + b1_ref[...]
    h = jnp.maximum(out1, 0.0)
    h2 = jnp.dot(h, w2_ref[...], preferred_element_type=jnp.float32)
    h2_ref[...] = h2
    g2_ref[...] = h2 * dinv


def _mid(acc1, h1, dinv, b1, W2):
    return pl.pallas_call(
        _mid_body,
        grid=(N // _RB,),
        in_specs=[pl.BlockSpec((NC, _RB, D), lambda i: (0, i, 0)),
                  pl.BlockSpec((_RB, D), lambda i: (i, 0)),
                  pl.BlockSpec((_RB, 16), lambda i: (i, 0)),
                  pl.BlockSpec((1, D), lambda i: (0, 0)),
                  pl.BlockSpec((D, D), lambda i: (0, 0))],
        out_specs=[pl.BlockSpec((_RB, D), lambda i: (i, 0)),
                   pl.BlockSpec((_RB, D), lambda i: (i, 0))],
        out_shape=[jax.ShapeDtypeStruct((N, D), jnp.float32),
                   jax.ShapeDtypeStruct((N, D), jnp.float32)],
    )(acc1, h1, dinv, b1, W2)


def _post_body(acc_ref, h2_ref, dinv_ref, b2_ref, out_ref):
    dinv = dinv_ref[...][:, :1]
    out_ref[...] = dinv * (acc_ref[0] + acc_ref[1]) \
        + (dinv * dinv) * h2_ref[...] + b2_ref[...]


def _post(acc2, h2, dinv, b2):
    return pl.pallas_call(
        _post_body,
        grid=(N // _RB,),
        in_specs=[pl.BlockSpec((NC, _RB, D), lambda i: (0, i, 0)),
                  pl.BlockSpec((_RB, D), lambda i: (i, 0)),
                  pl.BlockSpec((_RB, 16), lambda i: (i, 0)),
                  pl.BlockSpec((1, D), lambda i: (0, 0))],
        out_specs=pl.BlockSpec((_RB, D), lambda i: (i, 0)),
        out_shape=jax.ShapeDtypeStruct((N, D), jnp.float32),
    )(acc2, h2, dinv, b2)


def kernel(x, edge_index, W1, b1, W2, b2):
    ei = edge_index.astype(jnp.int32)
    pad = PAD_E - E
    src = jnp.concatenate([ei[0], jnp.zeros((pad,), jnp.int32)])
    dst = jnp.concatenate(
        [ei[1], N + (jnp.arange(pad, dtype=jnp.int32) % (ACC_ROWS - N))])
    dpad = DPAD_E - E
    dst3 = jnp.concatenate(
        [ei[1], N + (jnp.arange(dpad, dtype=jnp.int32) % (ACC_ROWS - N))]) \
        .reshape(NW, DSTEPS, DK)
    ones = jnp.ones((DK, D), jnp.float32)
    zerosD = jnp.zeros((ZROWS, D), jnp.float32)
    b1r = b1.reshape(1, D)
    b2r = b2.reshape(1, D)

    deg_parts = _deg_kernel(dst3, ones, zerosD)
    h1 = _matmul(x, W1)
    g1, dinv = _scale(deg_parts, h1)
    acc1 = _segsum_kernel(g1, src, dst, zerosD)
    g2, h2 = _mid(acc1, h1, dinv, b1r, W2)
    acc2 = _segsum_kernel(g2, src, dst, zerosD)
    return _post(acc2, h2, dinv, b2r)


# speedup vs baseline: 1.7745x; 1.0989x over previous
import functools

import jax
import jax.numpy as jnp
from jax import lax
from jax.experimental import pallas as pl
from jax.experimental.pallas import tpu as pltpu
from jax.experimental.pallas import tpu_sc as plsc

N = 10000
E = 320000
D = 128
NC = 2
NS = 16
NW = NC * NS
K = 80
S0 = 152
S1 = 100
C1OFF = NS * S0
PAD_E = NW * K * (S0 + S1) // 2
DK = 128
DSTEPS = 80
DPAD_E = NW * DK * DSTEPS
ACC_ROWS = 10240
ZROWS = ACC_ROWS // NS

def _sc_mesh():
    return plsc.VectorSubcoreMesh(core_axis_name="c", subcore_axis_name="s")


def _deg_body(dst_hbm, ones_hbm, zeros_hbm, out_hbm, dst_all, ones_v, acc_sh, sem):
    cid = lax.axis_index("c")
    sid = lax.axis_index("s")
    wid = cid * NS + sid
    pltpu.sync_copy(zeros_hbm, acc_sh.at[pl.ds(sid * ZROWS, ZROWS)])
    pltpu.sync_copy(ones_hbm, ones_v)
    pltpu.sync_copy(dst_hbm.at[wid], dst_all)
    plsc.subcore_barrier()

    @pl.loop(0, DSTEPS)
    def _(i):
        pltpu.sync_copy(ones_v, acc_sh.at[dst_all.at[i]], add=True)

    plsc.subcore_barrier()
    pltpu.sync_copy(acc_sh.at[pl.ds(sid * ZROWS, ZROWS)],
                    out_hbm.at[cid, pl.ds(sid * ZROWS, ZROWS)])


def _deg_kernel(dst3, ones, zerosD):
    return pl.kernel(
        _deg_body, mesh=_sc_mesh(),
        out_type=jax.ShapeDtypeStruct((NC, ACC_ROWS, D), jnp.float32),
        scratch_types=[
            pltpu.VMEM((DSTEPS, DK), jnp.int32),
            pltpu.VMEM((DK, D), jnp.float32),
            pltpu.VMEM_SHARED((ACC_ROWS, D), jnp.float32),
            pltpu.SemaphoreType.DMA,
        ],
    )(dst3, ones, zerosD)


def _segsum_body(g_hbm, src_hbm, dst_hbm, zeros_hbm, out_hbm,
                 src_a, dst_a, src_b, dst_b, rows_a, rows_b, acc_sh,
                 sem_a, sem_b):
    cid = lax.axis_index("c")
    sid = lax.axis_index("s")
    wid = cid * NS + sid
    pltpu.sync_copy(zeros_hbm, acc_sh.at[pl.ds(sid * ZROWS, ZROWS)])
    plsc.subcore_barrier()
    msteps = jnp.where(cid == 0, S0, S1)
    base = jnp.where(cid == 0, sid * S0, C1OFF + sid * S1) * K
    pltpu.sync_copy(src_hbm.at[pl.ds(base, K)], src_a)
    pltpu.sync_copy(dst_hbm.at[pl.ds(base, K)], dst_a)
    pltpu.async_copy(g_hbm.at[src_a], rows_a, sem_a)

    @pl.loop(0, S0, step=2)
    def _(i):
        @pl.when(i < msteps)
        def _():
            pltpu.sync_copy(src_hbm.at[pl.ds(base + (i + 1) * K, K)], src_b)
            pltpu.sync_copy(dst_hbm.at[pl.ds(base + (i + 1) * K, K)], dst_b)
            pltpu.async_copy(g_hbm.at[src_b], rows_b, sem_b)
            pltpu.make_async_copy(g_hbm.at[src_a], rows_a, sem_a).wait()
            pltpu.sync_copy(rows_a, acc_sh.at[dst_a], add=True)

            @pl.when(i + 2 < msteps)
            def _():
                pltpu.sync_copy(src_hbm.at[pl.ds(base + (i + 2) * K, K)], src_a)
                pltpu.sync_copy(dst_hbm.at[pl.ds(base + (i + 2) * K, K)], dst_a)
                pltpu.async_copy(g_hbm.at[src_a], rows_a, sem_a)

            pltpu.make_async_copy(g_hbm.at[src_b], rows_b, sem_b).wait()
            pltpu.sync_copy(rows_b, acc_sh.at[dst_b], add=True)

    plsc.subcore_barrier()
    pltpu.sync_copy(acc_sh.at[pl.ds(sid * ZROWS, ZROWS)],
                    out_hbm.at[cid, pl.ds(sid * ZROWS, ZROWS)])


def _segsum_kernel(g, src, dst, zerosD):
    return pl.kernel(
        _segsum_body, mesh=_sc_mesh(),
        out_type=jax.ShapeDtypeStruct((NC, ACC_ROWS, D), jnp.float32),
        scratch_types=[
            pltpu.VMEM((K,), jnp.int32),
            pltpu.VMEM((K,), jnp.int32),
            pltpu.VMEM((K,), jnp.int32),
            pltpu.VMEM((K,), jnp.int32),
            pltpu.VMEM((K, D), jnp.float32),
            pltpu.VMEM((K, D), jnp.float32),
            pltpu.VMEM_SHARED((ACC_ROWS, D), jnp.float32),
            pltpu.SemaphoreType.DMA,
            pltpu.SemaphoreType.DMA,
        ],
    )(g, src, dst, zerosD)


_RB = 2000


def _mm_body(x_ref, w_ref, o_ref):
    o_ref[...] = jnp.dot(x_ref[...], w_ref[...],
                         preferred_element_type=jnp.float32)


def _matmul(x, W):
    return pl.pallas_call(
        _mm_body,
        grid=(N // _RB,),
        in_specs=[pl.BlockSpec((_RB, D), lambda i: (i, 0)),
                  pl.BlockSpec((D, D), lambda i: (0, 0))],
        out_specs=pl.BlockSpec((_RB, D), lambda i: (i, 0)),
        out_shape=jax.ShapeDtypeStruct((N, D), jnp.float32),
    )(x, W)


def _scale_body(degp_ref, h_ref, g_ref, dinv_ref):
    deg = degp_ref[0][:, :16] + degp_ref[1][:, :16] + 1.0
    dinv = lax.rsqrt(deg)
    dinv_ref[...] = dinv
    g_ref[...] = h_ref[...] * dinv[:, :1]


def _scale(deg_parts, h):
    return pl.pallas_call(
        _scale_body,
        grid=(N // _RB,),
        in_specs=[pl.BlockSpec((NC, _RB, D), lambda i: (0, i, 0)),
                  pl.BlockSpec((_RB, D), lambda i: (i, 0))],
        out_specs=[pl.BlockSpec((_RB, D), lambda i: (i, 0)),
                   pl.BlockSpec((_RB, 16), lambda i: (i, 0))],
        out_shape=[jax.ShapeDtypeStruct((N, D), jnp.float32),
                   jax.ShapeDtypeStruct((N, 16), jnp.float32)],
    )(deg_parts, h)


def _mid_body(acc_ref, h1_ref, dinv_ref, b1_ref, w2_ref, g2_ref, h2_ref):
    dinv = dinv_ref[...][:, :1]
    out1 = dinv * (acc_ref[0] + acc_ref[1]) \
        + (dinv * dinv) * h1_ref[...] + b1_ref[...]
    h = jnp.maximum(out1, 0.0)
    h2 = jnp.dot(h, w2_ref[...], preferred_element_type=jnp.float32)
    h2_ref[...] = h2
    g2_ref[...] = h2 * dinv


def _mid(acc1, h1, dinv, b1, W2):
    return pl.pallas_call(
        _mid_body,
        grid=(N // _RB,),
        in_specs=[pl.BlockSpec((NC, _RB, D), lambda i: (0, i, 0)),
                  pl.BlockSpec((_RB, D), lambda i: (i, 0)),
                  pl.BlockSpec((_RB, 16), lambda i: (i, 0)),
                  pl.BlockSpec((1, D), lambda i: (0, 0)),
                  pl.BlockSpec((D, D), lambda i: (0, 0))],
        out_specs=[pl.BlockSpec((_RB, D), lambda i: (i, 0)),
                   pl.BlockSpec((_RB, D), lambda i: (i, 0))],
        out_shape=[jax.ShapeDtypeStruct((N, D), jnp.float32),
                   jax.ShapeDtypeStruct((N, D), jnp.float32)],
    )(acc1, h1, dinv, b1, W2)


def _post_body(acc_ref, h2_ref, dinv_ref, b2_ref, out_ref):
    dinv = dinv_ref[...][:, :1]
    out_ref[...] = dinv * (acc_ref[0] + acc_ref[1]) \
        + (dinv * dinv) * h2_ref[...] + b2_ref[...]


def _post(acc2, h2, dinv, b2):
    return pl.pallas_call(
        _post_body,
        grid=(N // _RB,),
        in_specs=[pl.BlockSpec((NC, _RB, D), lambda i: (0, i, 0)),
                  pl.BlockSpec((_RB, D), lambda i: (i, 0)),
                  pl.BlockSpec((_RB, 16), lambda i: (i, 0)),
                  pl.BlockSpec((1, D), lambda i: (0, 0))],
        out_specs=pl.BlockSpec((_RB, D), lambda i: (i, 0)),
        out_shape=jax.ShapeDtypeStruct((N, D), jnp.float32),
    )(acc2, h2, dinv, b2)


def kernel(x, edge_index, W1, b1, W2, b2):
    ei = edge_index.astype(jnp.int32)
    pad = PAD_E - E
    src = jnp.concatenate([ei[0], jnp.zeros((pad,), jnp.int32)])
    dst = jnp.concatenate(
        [ei[1], N + (jnp.arange(pad, dtype=jnp.int32) % (ACC_ROWS - N))])
    dpad = DPAD_E - E
    dst3 = jnp.concatenate(
        [ei[1], N + (jnp.arange(dpad, dtype=jnp.int32) % (ACC_ROWS - N))]) \
        .reshape(NW, DSTEPS, DK)
    ones = jnp.ones((DK, D), jnp.float32)
    zerosD = jnp.zeros((ZROWS, D), jnp.float32)
    b1r = b1.reshape(1, D)
    b2r = b2.reshape(1, D)

    deg_parts = _deg_kernel(dst3, ones, zerosD)
    h1 = _matmul(x, W1)
    g1, dinv = _scale(deg_parts, h1)
    acc1 = _segsum_kernel(g1, src, dst, zerosD)
    g2, h2 = _mid(acc1, h1, dinv, b1r, W2)
    acc2 = _segsum_kernel(g2, src, dst, zerosD)
    return _post(acc2, h2, dinv, b2r)
